# baseline (speedup 1.0000x reference)
import jax
import jax.numpy as jnp
from jax.experimental import pallas as pl
from jax.experimental.pallas import tpu as pltpu

_TOKENS = 16384
_DIM = 4096
_E = 64
_K = 8
_BR = 1024


def _gate_kernel(x1_ref, x2_ref, w_ref, fw_ref, idx_ref):
    half = _DIM // 2
    logits_t = jax.lax.dot_general(
        w_ref[:, :half], x1_ref[...],
        dimension_numbers=(((1,), (1,)), ((), ())),
        preferred_element_type=jnp.float32,
    ) + jax.lax.dot_general(
        w_ref[:, half:], x2_ref[...],
        dimension_numbers=(((1,), (1,)), ((), ())),
        preferred_element_type=jnp.float32,
    )

    erow = jax.lax.broadcasted_iota(jnp.int32, logits_t.shape, 0)
    neg_inf = jnp.float32(-jnp.inf)

    cur = logits_t
    sel = jnp.zeros(logits_t.shape, jnp.bool_)
    idx_rows = []
    mx = None
    denom = None
    for t in range(_K):
        m = jnp.max(cur, axis=0, keepdims=True)
        is_max = cur == m
        idx = jnp.min(jnp.where(is_max, erow, _E), axis=0, keepdims=True)
        chosen = erow == idx
        sel = sel | chosen
        cur = jnp.where(chosen, neg_inf, cur)
        idx_rows.append(idx)
        if t == 0:
            mx = m
            denom = jnp.ones(m.shape, jnp.float32)
        else:
            denom = denom + jnp.exp(m - mx)

    inv = 1.0 / denom
    fw_t = jnp.where(sel, jnp.exp(logits_t - mx) * inv, 0.0)
    idx_t = jnp.concatenate(idx_rows, axis=0)
    fw_ref[...] = fw_t.T
    idx_ref[...] = idx_t.T


@jax.jit
def kernel(x, W):
    grid = (_TOKENS // _BR,)
    fw, idx = pl.pallas_call(
        _gate_kernel,
        grid=grid,
        in_specs=[
            pl.BlockSpec((_BR, _DIM // 2), lambda i: (i, 0)),
            pl.BlockSpec((_BR, _DIM // 2), lambda i: (i, 1)),
            pl.BlockSpec((_E, _DIM), lambda i: (0, 0)),
        ],
        out_specs=[
            pl.BlockSpec((_BR, _E), lambda i: (i, 0)),
            pl.BlockSpec((_BR, _K), lambda i: (i, 0)),
        ],
        out_shape=[
            jax.ShapeDtypeStruct((_TOKENS, _E), jnp.float32),
            jax.ShapeDtypeStruct((_TOKENS, _K), jnp.int32),
        ],
        compiler_params=pltpu.CompilerParams(
            vmem_limit_bytes=100 * 1024 * 1024,
        ),
    )(x, x, W)
    return fw, idx
